# Initial kernel scaffold; baseline (speedup 1.0000x reference)
import jax
import jax.numpy as jnp
from jax.experimental import pallas as pl


def kernel(lidar, wind, w_wind1, b_wind1, w_wind2, b_wind2, w_wind3, b_wind3, conv1_w, conv1_b, conv2_w, conv2_b, fc1_w, fc1_b, fc2_w, fc2_b, fc3_w, fc3_b):
    raise NotImplementedError("write your pallas kernel here")


# speedup vs baseline: 1.4910x; 1.4910x over previous
import jax
import jax.numpy as jnp
import numpy as np
from jax.experimental import pallas as pl
from jax.experimental.pallas import tpu as pltpu

G = 250
CENTER = G // 2
K = 1000
PRED = 10

NSLAB = 10
SH = G // NSLAB
CPC = SH * G
NB = 8 * SH * G
COLS = 32 * G * G + 64
WPAD = CPC + 150
MAXA = ((COLS - WPAD) // 128) * 128


def _rasterize(lidar):
    B = lidar.shape[0]
    dtype = lidar.dtype
    angles = jnp.linspace(0.0, 2.0 * jnp.pi, 360)
    r = lidar
    valid_r = (r > 0.0) & (r < 1.0)
    n = jnp.floor((2.0 - r) * 500.0).astype(jnp.int32) + 1
    k = jnp.arange(K, dtype=jnp.int32)
    denom = jnp.maximum(n - 1, 1).astype(dtype)
    mag = r[..., None] + k.astype(dtype) * (2.0 - r)[..., None] / denom[..., None]
    step_ok = k[None, None, :] < n[..., None]
    x = mag * jnp.cos(angles)[None, :, None]
    y = mag * jnp.sin(angles)[None, :, None]
    xg = jnp.trunc(CENTER + x * 100.0).astype(jnp.int32)
    yg = jnp.trunc(CENTER + y * 100.0).astype(jnp.int32)
    inb = (xg >= 0) & (xg < G) & (yg >= 0) & (yg < G)
    valid = valid_r[..., None] & step_ok & inb
    idx = jnp.where(valid, yg * G + xg, G * G)
    flat = jnp.zeros((B, G * G + 1), dtype=dtype)
    flat = flat.at[jnp.arange(B)[:, None, None], idx].max(valid.astype(dtype))
    return flat[:, : G * G].reshape(B, G, G)


def _conv_fc_kernel(P_ref, w1_ref, b1_ref, w2_ref, b2_ref, fcw_hbm,
                    out_ref, a2_scr, wbuf, sems):
    yb = pl.program_id(0)
    c = pl.program_id(1)
    y0 = yb * SH

    def aligned_start(cc):
        start = cc * (G * G) + yb * CPC
        astart = jnp.minimum((start // 128) * 128, MAXA)
        return start, astart

    def start_fetch(cc):
        _, astart = aligned_start(cc)
        pltpu.make_async_copy(
            fcw_hbm.at[:, pl.ds(astart, WPAD)],
            wbuf.at[cc % 2],
            sems.at[cc % 2]).start()

    @pl.when(c == 0)
    def _prefetch_first():
        start_fetch(0)
        start_fetch(1)

    @pl.when((c > 0) & (c < 31))
    def _prefetch_next():
        start_fetch(c + 1)

    @pl.when(c == 0)
    def _compute_slab():
        Pt = P_ref[0]
        a1s = []
        for oc in range(16):
            acc = w1_ref[oc, 0, 0] * Pt[:, 0:SH + 2, 0:252]
            for dy in range(3):
                for dx in range(3):
                    if dy == 0 and dx == 0:
                        continue
                    acc = acc + w1_ref[oc, dy, dx] * Pt[:, dy:dy + SH + 2, dx:dx + 252]
            a1s.append(jax.nn.relu(acc + b1_ref[oc, 0]))
        a1 = jnp.stack(a1s)
        ya = y0 - 1 + jax.lax.broadcasted_iota(jnp.int32, (1, 1, SH + 2, 1), 2)
        xa = jax.lax.broadcasted_iota(jnp.int32, (1, 1, 1, 252), 3) - 1
        mask = (ya >= 0) & (ya < G) & (xa >= 0) & (xa < G)
        a1 = jnp.where(mask, a1, 0.0)
        acc2 = jnp.zeros((32, NB), jnp.float32)
        for dy in range(3):
            for dx in range(3):
                sl = a1[:, :, dy:dy + SH, dx:dx + G].reshape(16, NB)
                acc2 = acc2 + jax.lax.dot_general(
                    w2_ref[:, :, dy, dx], sl,
                    (((1,), (0,)), ((), ())),
                    preferred_element_type=jnp.float32)
        a2 = jax.nn.relu(acc2 + b2_ref[:, 0:1])
        a2 = a2.reshape(32, 8, CPC)
        a2_scr[...] = jnp.pad(a2, ((0, 0), (0, 0), (0, WPAD - CPC)))

    start, astart = aligned_start(c)
    pltpu.make_async_copy(
        fcw_hbm.at[:, pl.ds(astart, WPAD)],
        wbuf.at[c % 2],
        sems.at[c % 2]).wait()
    a2c = a2_scr[pl.ds(c, 1)].reshape(8, WPAD)
    a2r = pltpu.roll(a2c, start - astart, axis=1)
    wblk = wbuf[pl.ds(c % 2, 1)].reshape(64, WPAD)
    contrib = jax.lax.dot_general(
        a2r, wblk,
        (((1,), (1,)), ((), ())),
        preferred_element_type=jnp.float32)

    @pl.when(c == 0)
    def _init():
        out_ref[0] = contrib

    @pl.when(c != 0)
    def _acc():
        out_ref[0] = out_ref[0] + contrib


def _head_kernel(part_ref, wind_ref, ww1_ref, bw1_ref, ww2_ref, bw2_ref,
                 ww3_ref, bw3_ref, fw_wind_ref, fb1_ref, w2_ref, b2_ref,
                 w3_ref, b3_ref, out_ref):
    def lin(x, w_ref, b_ref):
        return jax.lax.dot_general(
            x, w_ref[...], (((1,), (1,)), ((), ())),
            preferred_element_type=jnp.float32) + b_ref[...]

    wnd = jax.nn.relu(lin(wind_ref[...], ww1_ref, bw1_ref))
    wnd = jax.nn.relu(lin(wnd, ww2_ref, bw2_ref))
    wnd = jax.nn.relu(lin(wnd, ww3_ref, bw3_ref))
    himg = jnp.sum(part_ref[...], axis=0)
    x = jax.nn.relu(himg + lin(wnd, fw_wind_ref, fb1_ref))
    x = jax.nn.relu(lin(x, w2_ref, b2_ref))
    out_ref[...] = lin(x, w3_ref, b3_ref)


def kernel(lidar, wind, w_wind1, b_wind1, w_wind2, b_wind2, w_wind3, b_wind3,
           conv1_w, conv1_b, conv2_w, conv2_b, fc1_w, fc1_b, fc2_w, fc2_b,
           fc3_w, fc3_b):
    B = lidar.shape[0]
    img = _rasterize(lidar)
    P = jnp.pad(img, ((0, 0), (2, 2), (2, 2)))
    P_slabs = jnp.stack([P[:, s * SH: s * SH + SH + 4, :]
                         for s in range(NSLAB)])

    partial = pl.pallas_call(
        _conv_fc_kernel,
        grid=(NSLAB, 32),
        in_specs=[
            pl.BlockSpec((1, B, SH + 4, G + 4), lambda yb, c: (yb, 0, 0, 0)),
            pl.BlockSpec((16, 3, 3), lambda yb, c: (0, 0, 0)),
            pl.BlockSpec((16, 1), lambda yb, c: (0, 0)),
            pl.BlockSpec((32, 16, 3, 3), lambda yb, c: (0, 0, 0, 0)),
            pl.BlockSpec((32, 1), lambda yb, c: (0, 0)),
            pl.BlockSpec(memory_space=pl.ANY),
        ],
        out_specs=pl.BlockSpec((1, B, 64), lambda yb, c: (yb, 0, 0)),
        out_shape=jax.ShapeDtypeStruct((NSLAB, B, 64), jnp.float32),
        scratch_shapes=[pltpu.VMEM((32, B, WPAD), jnp.float32),
                        pltpu.VMEM((2, 64, WPAD), jnp.float32),
                        pltpu.SemaphoreType.DMA((2,))],
        compiler_params=pltpu.CompilerParams(
            dimension_semantics=("parallel", "arbitrary")),
    )(P_slabs, conv1_w.reshape(16, 3, 3), conv1_b.reshape(16, 1),
      conv2_w, conv2_b.reshape(32, 1), fc1_w)

    fw_wind = jax.lax.slice(fc1_w, (0, G * G * 32), (64, G * G * 32 + 64))

    out = pl.pallas_call(
        _head_kernel,
        grid=(1,),
        in_specs=[
            pl.BlockSpec((NSLAB, B, 64), lambda i: (0, 0, 0)),
            pl.BlockSpec((B, 2), lambda i: (0, 0)),
            pl.BlockSpec((64, 2), lambda i: (0, 0)),
            pl.BlockSpec((1, 64), lambda i: (0, 0)),
            pl.BlockSpec((64, 64), lambda i: (0, 0)),
            pl.BlockSpec((1, 64), lambda i: (0, 0)),
            pl.BlockSpec((64, 64), lambda i: (0, 0)),
            pl.BlockSpec((1, 64), lambda i: (0, 0)),
            pl.BlockSpec((64, 64), lambda i: (0, 0)),
            pl.BlockSpec((1, 64), lambda i: (0, 0)),
            pl.BlockSpec((32, 64), lambda i: (0, 0)),
            pl.BlockSpec((1, 32), lambda i: (0, 0)),
            pl.BlockSpec((200, 32), lambda i: (0, 0)),
            pl.BlockSpec((1, 200), lambda i: (0, 0)),
        ],
        out_specs=pl.BlockSpec((B, 200), lambda i: (0, 0)),
        out_shape=jax.ShapeDtypeStruct((B, 200), jnp.float32),
    )(partial, wind, w_wind1, b_wind1.reshape(1, 64), w_wind2,
      b_wind2.reshape(1, 64), w_wind3, b_wind3.reshape(1, 64), fw_wind,
      fc1_b.reshape(1, 64), fc2_w, fc2_b.reshape(1, 32), fc3_w,
      fc3_b.reshape(1, 200))

    return out.reshape(B, PRED, PRED, 2)
